# baseline (speedup 1.0000x reference)
import functools

import jax
import jax.numpy as jnp
from jax import lax
from jax.experimental import pallas as pl
from jax.experimental.pallas import tpu as pltpu
from jax.experimental.pallas import tpu_sc as plsc

N_TOK = 8192
K_CODES = 8192
D = 256
BETA = 0.25

BN = 512
NB = N_TOK // BN


def _mega_body(zn_ref, en_ref, kio_ref, z_ref, e_ref,
               idx_ref, oh_ref, loss_ref, ppl_ref,
               cnt_ref, acc_ref):
    i = pl.program_id(0)

    m2 = jax.lax.dot_general(
        z_ref[...], e_ref[...],
        dimension_numbers=(((1,), (1,)), ((), ())),
        preferred_element_type=jnp.float32)
    s = zn_ref[...] + en_ref[...]
    d = s - m2

    bmin = jnp.min(d, axis=1, keepdims=True)
    kiota = kio_ref[...]
    bargf = jnp.min(jnp.where(d == bmin, kiota, 1e9),
                    axis=1, keepdims=True)
    idx_ref[...] = bargf.astype(jnp.int32)

    oh = jnp.where(kiota == bargf, 1.0, 0.0)
    oh_ref[...] = oh

    @pl.when(i == 0)
    def _init():
        cnt_ref[...] = jnp.zeros((1, K_CODES), jnp.float32)
        acc_ref[0, 0] = 0.0

    ones_row = jnp.full((1, BN), 1.0, jnp.float32)
    cnt_ref[...] += jax.lax.dot_general(
        ones_row, oh,
        dimension_numbers=(((1,), (0,)), ((), ())),
        preferred_element_type=jnp.float32)

    acc_ref[0, 0] += jnp.sum(bmin)

    @pl.when(i == NB - 1)
    def _emit_scalars():
        mloss = acc_ref[0, 0] * (1.0 / (N_TOK * D))
        loss_ref[0, 0] = mloss + BETA * mloss
        e_mean = cnt_ref[...] * (1.0 / N_TOK)
        ent = jnp.sum(e_mean * jnp.log(e_mean + 1e-10))
        ppl_ref[0, 0] = jnp.exp(-ent)


def _make_sc_gather():
    info = plsc.get_sparse_core_info()
    nw = info.num_cores * info.num_subcores
    b_per_w = N_TOK // nw
    mesh = plsc.VectorSubcoreMesh(core_axis_name="c", subcore_axis_name="s")

    @functools.partial(
        pl.kernel, mesh=mesh,
        out_type=jax.ShapeDtypeStruct((N_TOK, D), jnp.float32),
        scratch_types=[
            pltpu.VMEM((b_per_w,), jnp.int32),
            pltpu.VMEM((b_per_w, D), jnp.float32),
            pltpu.SemaphoreType.DMA,
        ],
    )
    def gather_k(table_hbm, idx_hbm, out_hbm, idx_v, rows_v, sem):
        wid = lax.axis_index("s") * info.num_cores + lax.axis_index("c")
        base = wid * b_per_w
        pltpu.sync_copy(idx_hbm.at[pl.ds(base, b_per_w)], idx_v)
        pltpu.async_copy(table_hbm.at[idx_v], rows_v, sem).wait()
        pltpu.sync_copy(rows_v, out_hbm.at[pl.ds(base, b_per_w)])

    return gather_k


_gather_fn = None


def _sc_gather(table, idx):
    global _gather_fn
    if _gather_fn is None:
        _gather_fn = _make_sc_gather()
    return _gather_fn(table, idx)


def kernel(z, embedding_weight):
    zt = jnp.transpose(z, (0, 2, 3, 1))
    zf2 = (zt * 2.0).reshape(N_TOK, D)
    zn = jnp.sum((zf2 * 0.5) ** 2, axis=1)
    en = jnp.sum(embedding_weight ** 2, axis=1)

    idx2, onehot, loss, ppl = pl.pallas_call(
        _mega_body,
        grid=(NB,),
        in_specs=[
            pl.BlockSpec((BN, 1), lambda i: (i, 0)),
            pl.BlockSpec((1, K_CODES), lambda i: (0, 0)),
            pl.BlockSpec((1, K_CODES), lambda i: (0, 0)),
            pl.BlockSpec((BN, D), lambda i: (i, 0)),
            pl.BlockSpec((K_CODES, D), lambda i: (0, 0)),
        ],
        out_specs=[
            pl.BlockSpec((BN, 1), lambda i: (i, 0)),
            pl.BlockSpec((BN, K_CODES), lambda i: (i, 0)),
            pl.BlockSpec(memory_space=pltpu.SMEM),
            pl.BlockSpec(memory_space=pltpu.SMEM),
        ],
        out_shape=[
            jax.ShapeDtypeStruct((N_TOK, 1), jnp.int32),
            jax.ShapeDtypeStruct((N_TOK, K_CODES), jnp.float32),
            jax.ShapeDtypeStruct((1, 1), jnp.float32),
            jax.ShapeDtypeStruct((1, 1), jnp.float32),
        ],
        scratch_shapes=[
            pltpu.VMEM((1, K_CODES), jnp.float32),
            pltpu.SMEM((1, 1), jnp.float32),
        ],
    )(zn.reshape(N_TOK, 1), en.reshape(1, K_CODES),
      jnp.arange(K_CODES, dtype=jnp.float32).reshape(1, K_CODES),
      zf2, embedding_weight)

    zq = _sc_gather(embedding_weight, idx2.reshape(N_TOK))

    z_q = zq.reshape(8, 32, 32, 256).transpose(0, 3, 1, 2)
    return (z_q, loss.reshape(()), ppl.reshape(()), onehot, idx2)


# speedup vs baseline: 1.1273x; 1.1273x over previous
import functools

import jax
import jax.numpy as jnp
from jax import lax
from jax.experimental import pallas as pl
from jax.experimental.pallas import tpu as pltpu
from jax.experimental.pallas import tpu_sc as plsc

N_TOK = 8192
K_CODES = 8192
D = 256
BETA = 0.25

BN = 512
NB = N_TOK // BN


def _mega_body(zn_ref, en_ref, kio_ref, z_ref, e_ref,
               idx_ref, oh_ref, loss_ref, ppl_ref,
               cnt_ref, acc_ref):
    i = pl.program_id(0)

    m = jax.lax.dot_general(
        z_ref[...], e_ref[...],
        dimension_numbers=(((1,), (1,)), ((), ())),
        preferred_element_type=jnp.float32)
    s = zn_ref[...] + en_ref[...]
    d = s - 2.0 * m

    bmin = jnp.min(d, axis=1, keepdims=True)
    kiota = kio_ref[...]
    bargf = jnp.min(jnp.where(d == bmin, kiota, 1e9),
                    axis=1, keepdims=True)
    idx_ref[...] = bargf.astype(jnp.int32)

    oh = jnp.where(kiota == bargf, 1.0, 0.0)
    oh_ref[...] = oh

    @pl.when(i == 0)
    def _init():
        cnt_ref[...] = jnp.zeros((1, K_CODES), jnp.float32)
        acc_ref[0, 0] = 0.0

    ones_row = jnp.full((1, BN), 1.0, jnp.float32)
    cnt_ref[...] += jax.lax.dot_general(
        ones_row, oh,
        dimension_numbers=(((1,), (0,)), ((), ())),
        preferred_element_type=jnp.float32)

    acc_ref[0, 0] += jnp.sum(bmin)

    @pl.when(i == NB - 1)
    def _emit_scalars():
        mloss = acc_ref[0, 0] * (1.0 / (N_TOK * D))
        loss_ref[0, 0] = mloss + BETA * mloss
        e_mean = cnt_ref[...] * (1.0 / N_TOK)
        ent = jnp.sum(e_mean * jnp.log(e_mean + 1e-10))
        ppl_ref[0, 0] = jnp.exp(-ent)


def _make_sc_gather():
    info = plsc.get_sparse_core_info()
    nw = info.num_cores * info.num_subcores
    b_per_w = N_TOK // nw
    mesh = plsc.VectorSubcoreMesh(core_axis_name="c", subcore_axis_name="s")

    chunk = b_per_w // 2

    @functools.partial(
        pl.kernel, mesh=mesh,
        out_type=jax.ShapeDtypeStruct((N_TOK, D), jnp.float32),
        scratch_types=[
            pltpu.VMEM((b_per_w,), jnp.int32),
            pltpu.VMEM((chunk, D), jnp.float32),
            pltpu.VMEM((chunk, D), jnp.float32),
            pltpu.SemaphoreType.DMA,
            pltpu.SemaphoreType.DMA,
            pltpu.SemaphoreType.DMA,
        ],
    )
    def gather_k(table_hbm, idx_hbm, out_hbm, idx_v, rows_a, rows_b,
                 gsem_a, gsem_b, wsem):
        wid = lax.axis_index("s") * info.num_cores + lax.axis_index("c")
        base = wid * b_per_w
        pltpu.sync_copy(idx_hbm.at[pl.ds(base, b_per_w)], idx_v)
        ga = pltpu.async_copy(
            table_hbm.at[idx_v.at[pl.ds(0, chunk)]], rows_a, gsem_a)
        gb = pltpu.async_copy(
            table_hbm.at[idx_v.at[pl.ds(chunk, chunk)]], rows_b, gsem_b)
        ga.wait()
        wa = pltpu.async_copy(
            rows_a, out_hbm.at[pl.ds(base, chunk)], wsem)
        gb.wait()
        wb = pltpu.async_copy(
            rows_b, out_hbm.at[pl.ds(base + chunk, chunk)], wsem)
        wa.wait()
        wb.wait()

    return gather_k


_gather_fn = None


def _sc_gather(table, idx):
    global _gather_fn
    if _gather_fn is None:
        _gather_fn = _make_sc_gather()
    return _gather_fn(table, idx)


def kernel(z, embedding_weight):
    zt = jnp.transpose(z, (0, 2, 3, 1))
    zf = zt.reshape(N_TOK, D)
    zn = jnp.sum(zf ** 2, axis=1)
    en = jnp.sum(embedding_weight ** 2, axis=1)

    idx2, onehot, loss, ppl = pl.pallas_call(
        _mega_body,
        grid=(NB,),
        in_specs=[
            pl.BlockSpec((BN, 1), lambda i: (i, 0)),
            pl.BlockSpec((1, K_CODES), lambda i: (0, 0)),
            pl.BlockSpec((1, K_CODES), lambda i: (0, 0)),
            pl.BlockSpec((BN, D), lambda i: (i, 0)),
            pl.BlockSpec((K_CODES, D), lambda i: (0, 0)),
        ],
        out_specs=[
            pl.BlockSpec((BN, 1), lambda i: (i, 0)),
            pl.BlockSpec((BN, K_CODES), lambda i: (i, 0)),
            pl.BlockSpec(memory_space=pltpu.SMEM),
            pl.BlockSpec(memory_space=pltpu.SMEM),
        ],
        out_shape=[
            jax.ShapeDtypeStruct((N_TOK, 1), jnp.int32),
            jax.ShapeDtypeStruct((N_TOK, K_CODES), jnp.float32),
            jax.ShapeDtypeStruct((1, 1), jnp.float32),
            jax.ShapeDtypeStruct((1, 1), jnp.float32),
        ],
        scratch_shapes=[
            pltpu.VMEM((1, K_CODES), jnp.float32),
            pltpu.SMEM((1, 1), jnp.float32),
        ],
    )(zn.reshape(N_TOK, 1), en.reshape(1, K_CODES),
      jnp.arange(K_CODES, dtype=jnp.float32).reshape(1, K_CODES),
      zf, embedding_weight)

    zq = _sc_gather(embedding_weight, idx2.reshape(N_TOK))

    z_q = zq.reshape(8, 32, 32, 256).transpose(0, 3, 1, 2)
    return (z_q, loss.reshape(()), ppl.reshape(()), onehot, idx2)
